# Initial kernel scaffold; baseline (speedup 1.0000x reference)
import jax
import jax.numpy as jnp
from jax.experimental import pallas as pl


def kernel(x, transitions, transition_probs, initial_probs):
    raise NotImplementedError("write your pallas kernel here")


# speedup vs baseline: 12.3801x; 12.3801x over previous
import functools

import jax
import jax.numpy as jnp
from jax import lax
from jax.experimental import pallas as pl
from jax.experimental.pallas import tpu as pltpu
from jax.experimental.pallas import tpu_sc as plsc

NSTATES = 2048
NPDFS = 512
NTRANS = 32768
BATCH = 32
TFRAMES = 300
TPAD = 320

NC, NSUB, L = 2, 16, 16
G_TRANS = NTRANS // L
G_STATE = NSTATES // L
G_PDF = NPDFS // L
UNROLL = 8


def _sc_forward(x, packed, tp, init):
  mesh = plsc.VectorSubcoreMesh(core_axis_name="c", subcore_axis_name="s")

  @functools.partial(
      pl.kernel,
      mesh=mesh,
      out_type=jax.ShapeDtypeStruct((BATCH, TPAD), jnp.float32),
      compiler_params=pltpu.CompilerParams(needs_layout_passes=False),
      scratch_types=[
          pltpu.VMEM((NTRANS,), jnp.int32),
          pltpu.VMEM((NTRANS,), jnp.float32),
          pltpu.VMEM((NSTATES,), jnp.float32),
          pltpu.VMEM((NSTATES,), jnp.float32),
          pltpu.VMEM((NPDFS,), jnp.float32),
          pltpu.VMEM((NPDFS,), jnp.float32),
          pltpu.VMEM((TPAD,), jnp.float32),
          pltpu.SemaphoreType.DMA,
          pltpu.SemaphoreType.DMA,
          pltpu.SemaphoreType.DMA,
          pltpu.SemaphoreType.DMA,
          pltpu.SemaphoreType.DMA,
      ],
  )
  def fwd(x_hbm, pk_hbm, tp_hbm, init_hbm, out_hbm,
          pk_v, tp_v, a_ping, a_pong, px0, px1, cs_v, s0, s1, sp, st, si):
    b = lax.axis_index("s") * NC + lax.axis_index("c")
    ZERO = jnp.zeros((L,), jnp.float32)
    ONE = jnp.full((L,), 1.0, jnp.float32)
    lane = lax.iota(jnp.int32, L)

    cp_pk = pltpu.make_async_copy(pk_hbm, pk_v, sp)
    cp_tp = pltpu.make_async_copy(tp_hbm, tp_v, st)
    cp_in = pltpu.make_async_copy(init_hbm, a_ping, si)
    cp_pk.start()
    cp_tp.start()
    cp_in.start()
    cp_x0 = pltpu.make_async_copy(x_hbm.at[b, 0], px0, s0)
    cp_x1 = pltpu.make_async_copy(x_hbm.at[b, 1], px1, s1)
    cp_x0.start()
    cp_x1.start()

    def _zero(j, _):
      a_pong[pl.ds(j * L, L)] = ZERO
      return 0
    lax.fori_loop(0, G_STATE, _zero, 0)

    def _ones(j, _):
      cs_v[pl.ds(j * L, L)] = ONE
      return 0
    lax.fori_loop(0, TPAD // L, _ones, 0)

    cp_pk.wait()
    cp_tp.wait()
    cp_in.wait()

    def exp_pass(px):
      def eg(j, _):
        for u in range(4):
          off = j * (4 * L) + u * L
          px[pl.ds(off, L)] = jnp.exp(px[pl.ds(off, L)])
        return 0
      lax.fori_loop(0, G_PDF // 4, eg, 0)

    def frame_step(src_a, dst_a, px):
      def tgrp(j, _):
        base = j * (UNROLL * L)
        for u in range(UNROLL):
          off = base + u * L
          pk = pk_v[pl.ds(off, L)]
          w = tp_v[pl.ds(off, L)]
          s = pk >> 20
          d = (pk >> 9) & 2047
          p = pk & 511
          a = plsc.load_gather(src_a, [s])
          e = plsc.load_gather(px, [p])
          plsc.addupdate_scatter(dst_a, [d], a * w * e)
        return 0
      lax.fori_loop(0, G_TRANS // UNROLL, tgrp, 0)

      def rgrp(j, acc):
        for u in range(4):
          acc = acc + dst_a[pl.ds(j * (4 * L) + u * L, L)]
        return acc
      acc = lax.fori_loop(0, G_STATE // 4, rgrp, ZERO)
      c = jnp.sum(acc)
      inv = ONE / lax.broadcast_in_dim(c, (L,), ())

      def ngrp(j, _):
        for u in range(4):
          off = j * (4 * L) + u * L
          dst_a[pl.ds(off, L)] = dst_a[pl.ds(off, L)] * inv
          src_a[pl.ds(off, L)] = ZERO
        return 0
      lax.fori_loop(0, G_STATE // 4, ngrp, 0)
      return c

    def it(i, cvec):
      t0 = 2 * i
      pltpu.make_async_copy(x_hbm.at[b, t0], px0, s0).wait()
      exp_pass(px0)
      c0 = frame_step(a_ping, a_pong, px0)

      @pl.when(i < TFRAMES // 2 - 1)
      def _():
        pltpu.make_async_copy(x_hbm.at[b, t0 + 2], px0, s0).start()

      pltpu.make_async_copy(x_hbm.at[b, t0 + 1], px1, s1).wait()
      exp_pass(px1)
      c1 = frame_step(a_pong, a_ping, px1)

      @pl.when(i < TFRAMES // 2 - 1)
      def _():
        pltpu.make_async_copy(x_hbm.at[b, t0 + 3], px1, s1).start()

      cvec = jnp.where(lane == (t0 & 15), c0, cvec)
      cvec = jnp.where(lane == ((t0 + 1) & 15), c1, cvec)

      @pl.when((i & 7) == 7)
      def _():
        cs_v[pl.ds((i >> 3) * 16, 16)] = cvec

      return jnp.where((i & 7) == 7, ONE, cvec)

    cvec = lax.fori_loop(0, TFRAMES // 2, it, ONE)
    cs_v[pl.ds((TFRAMES // 16) * 16, L)] = cvec
    pltpu.sync_copy(cs_v, out_hbm.at[b])

  return fwd(x, packed, tp, init)


def _logsum(cs, batch):
  def body(cs_ref, o_ref):
    o_ref[0, 0] = jnp.sum(jnp.log(cs_ref[...])) * (1.0 / batch)

  out = pl.pallas_call(
      body,
      out_shape=jax.ShapeDtypeStruct((1, 1), jnp.float32),
      out_specs=pl.BlockSpec(memory_space=pltpu.SMEM),
  )(cs)
  return out[0, 0]


def kernel(x, transitions, transition_probs, initial_probs):
  src = transitions[:, 0]
  dst = transitions[:, 1]
  pdf = transitions[:, 2]
  packed = (src << 20) | (dst << 9) | pdf
  cs = _sc_forward(x, packed, transition_probs, initial_probs)
  return _logsum(cs, x.shape[0])


# speedup vs baseline: 39.8661x; 3.2202x over previous
import functools

import jax
import jax.numpy as jnp
from jax import lax
from jax.experimental import pallas as pl
from jax.experimental.pallas import tpu as pltpu
from jax.experimental.pallas import tpu_sc as plsc

NSTATES = 2048
NPDFS = 512
NTRANS = 32768
BATCH = 32
TFRAMES = 300
TPAD = 320

NC, NSUB, L = 2, 16, 16
G_TRANS = NTRANS // L
G_STATE = NSTATES // L
G_PDF = NPDFS // L
UNROLL = 8


def _sc_forward(x, packed, tp, init):
  mesh = plsc.VectorSubcoreMesh(core_axis_name="c", subcore_axis_name="s")

  @functools.partial(
      pl.kernel,
      mesh=mesh,
      out_type=jax.ShapeDtypeStruct((BATCH, TPAD), jnp.float32),
      compiler_params=pltpu.CompilerParams(needs_layout_passes=False),
      scratch_types=[
          pltpu.VMEM((NTRANS,), jnp.int32),
          pltpu.VMEM((NTRANS,), jnp.float32),
          pltpu.VMEM((NSTATES,), jnp.float32),
          pltpu.VMEM((NSTATES,), jnp.float32),
          pltpu.VMEM((NPDFS,), jnp.float32),
          pltpu.VMEM((NPDFS,), jnp.float32),
          pltpu.VMEM((TPAD,), jnp.float32),
          pltpu.SemaphoreType.DMA,
          pltpu.SemaphoreType.DMA,
          pltpu.SemaphoreType.DMA,
          pltpu.SemaphoreType.DMA,
          pltpu.SemaphoreType.DMA,
      ],
  )
  def fwd(x_hbm, pk_hbm, tp_hbm, init_hbm, out_hbm,
          pk_v, tp_v, a_ping, a_pong, px0, px1, cs_v, s0, s1, sp, st, si):
    b = lax.axis_index("s") * NC + lax.axis_index("c")
    ZERO = jnp.zeros((L,), jnp.float32)
    ONE = jnp.full((L,), 1.0, jnp.float32)
    lane = lax.iota(jnp.int32, L)

    cp_pk = pltpu.make_async_copy(pk_hbm, pk_v, sp)
    cp_tp = pltpu.make_async_copy(tp_hbm, tp_v, st)
    cp_in = pltpu.make_async_copy(init_hbm, a_ping, si)
    cp_pk.start()
    cp_tp.start()
    cp_in.start()
    cp_x0 = pltpu.make_async_copy(x_hbm.at[b, 0], px0, s0)
    cp_x1 = pltpu.make_async_copy(x_hbm.at[b, 1], px1, s1)
    cp_x0.start()
    cp_x1.start()

    def _zero(j, _):
      a_pong[pl.ds(j * L, L)] = ZERO
      return 0
    lax.fori_loop(0, G_STATE, _zero, 0)

    def _ones(j, _):
      cs_v[pl.ds(j * L, L)] = ONE
      return 0
    lax.fori_loop(0, TPAD // L, _ones, 0)

    cp_pk.wait()
    cp_tp.wait()
    cp_in.wait()

    def exp_pass(px):
      @plsc.parallel_loop(0, G_PDF, unroll=4)
      def _(j):
        px[pl.ds(j * L, L)] = jnp.exp(px[pl.ds(j * L, L)])

    def frame_step(src_a, dst_a, px):
      @plsc.parallel_loop(0, G_TRANS, unroll=UNROLL)
      def _(j):
        off = j * L
        pk = pk_v[pl.ds(off, L)]
        w = tp_v[pl.ds(off, L)]
        s = pk >> 20
        d = (pk >> 9) & 2047
        p = pk & 511
        a = plsc.load_gather(src_a, [s])
        e = plsc.load_gather(px, [p])
        plsc.addupdate_scatter(dst_a, [d], a * w * e)

      def acc_body(j, acc):
        a0, a1 = acc
        v = dst_a[pl.ds(j * L, L)]
        return (a1, a0 + v)
      accs = plsc.parallel_loop(
          0, G_STATE, unroll=4, carry=(ZERO, ZERO))(acc_body)
      c = jnp.sum(accs[0] + accs[1])
      inv = ONE / lax.broadcast_in_dim(c, (L,), ())

      @plsc.parallel_loop(0, G_STATE, unroll=4)
      def _(j):
        off = j * L
        dst_a[pl.ds(off, L)] = dst_a[pl.ds(off, L)] * inv
        src_a[pl.ds(off, L)] = ZERO
      return c

    def it(i, cvec):
      t0 = 2 * i
      pltpu.make_async_copy(x_hbm.at[b, t0], px0, s0).wait()
      exp_pass(px0)
      c0 = frame_step(a_ping, a_pong, px0)

      @pl.when(i < TFRAMES // 2 - 1)
      def _():
        pltpu.make_async_copy(x_hbm.at[b, t0 + 2], px0, s0).start()

      pltpu.make_async_copy(x_hbm.at[b, t0 + 1], px1, s1).wait()
      exp_pass(px1)
      c1 = frame_step(a_pong, a_ping, px1)

      @pl.when(i < TFRAMES // 2 - 1)
      def _():
        pltpu.make_async_copy(x_hbm.at[b, t0 + 3], px1, s1).start()

      cvec = jnp.where(lane == (t0 & 15), c0, cvec)
      cvec = jnp.where(lane == ((t0 + 1) & 15), c1, cvec)

      @pl.when((i & 7) == 7)
      def _():
        cs_v[pl.ds((i >> 3) * 16, 16)] = cvec

      return jnp.where((i & 7) == 7, ONE, cvec)

    cvec = lax.fori_loop(0, TFRAMES // 2, it, ONE)
    cs_v[pl.ds((TFRAMES // 16) * 16, L)] = cvec
    pltpu.sync_copy(cs_v, out_hbm.at[b])

  return fwd(x, packed, tp, init)


def _logsum(cs, batch):
  def body(cs_ref, o_ref):
    o_ref[0, 0] = jnp.sum(jnp.log(cs_ref[...])) * (1.0 / batch)

  out = pl.pallas_call(
      body,
      out_shape=jax.ShapeDtypeStruct((1, 1), jnp.float32),
      out_specs=pl.BlockSpec(memory_space=pltpu.SMEM),
  )(cs)
  return out[0, 0]


def kernel(x, transitions, transition_probs, initial_probs):
  src = transitions[:, 0]
  dst = transitions[:, 1]
  pdf = transitions[:, 2]
  packed = (src << 20) | (dst << 9) | pdf
  cs = _sc_forward(x, packed, transition_probs, initial_probs)
  return _logsum(cs, x.shape[0])


# speedup vs baseline: 44.8999x; 1.1263x over previous
import functools

import jax
import jax.numpy as jnp
from jax import lax
from jax.experimental import pallas as pl
from jax.experimental.pallas import tpu as pltpu
from jax.experimental.pallas import tpu_sc as plsc

NSTATES = 2048
NPDFS = 512
NTRANS = 32768
BATCH = 32
TFRAMES = 300
TPAD = 320

NC, NSUB, L = 2, 16, 16
G_TRANS = NTRANS // L
G_STATE = NSTATES // L
G_PDF = NPDFS // L
UNROLL = 8


def _sc_forward(x, packed, tp, init):
  mesh = plsc.VectorSubcoreMesh(core_axis_name="c", subcore_axis_name="s")

  @functools.partial(
      pl.kernel,
      mesh=mesh,
      out_type=jax.ShapeDtypeStruct((BATCH, TPAD), jnp.float32),
      compiler_params=pltpu.CompilerParams(needs_layout_passes=False),
      scratch_types=[
          pltpu.VMEM((NTRANS,), jnp.int32),
          pltpu.VMEM((NTRANS,), jnp.bfloat16),
          pltpu.VMEM((NSTATES,), jnp.float32),
          pltpu.VMEM((NSTATES,), jnp.float32),
          pltpu.VMEM((NPDFS,), jnp.float32),
          pltpu.VMEM((NPDFS,), jnp.float32),
          pltpu.VMEM((TPAD,), jnp.float32),
          pltpu.SemaphoreType.DMA,
          pltpu.SemaphoreType.DMA,
          pltpu.SemaphoreType.DMA,
          pltpu.SemaphoreType.DMA,
          pltpu.SemaphoreType.DMA,
      ],
  )
  def fwd(x_hbm, pk_hbm, tp_hbm, init_hbm, out_hbm,
          pk_v, tp_v, a_ping, a_pong, px0, px1, cs_v, s0, s1, sp, st, si):
    b = lax.axis_index("s") * NC + lax.axis_index("c")
    ZERO = jnp.zeros((L,), jnp.float32)
    ONE = jnp.full((L,), 1.0, jnp.float32)
    lane = lax.iota(jnp.int32, L)

    cp_pk = pltpu.make_async_copy(pk_hbm, pk_v, sp)
    cp_tp = pltpu.make_async_copy(tp_hbm, tp_v, st)
    cp_in = pltpu.make_async_copy(init_hbm, a_ping, si)
    cp_pk.start()
    cp_tp.start()
    cp_in.start()
    cp_x0 = pltpu.make_async_copy(x_hbm.at[b, 0], px0, s0)
    cp_x1 = pltpu.make_async_copy(x_hbm.at[b, 1], px1, s1)
    cp_x0.start()
    cp_x1.start()

    def _zero(j, _):
      a_pong[pl.ds(j * L, L)] = ZERO
      return 0
    lax.fori_loop(0, G_STATE, _zero, 0)

    def _ones(j, _):
      cs_v[pl.ds(j * L, L)] = ONE
      return 0
    lax.fori_loop(0, TPAD // L, _ones, 0)

    cp_pk.wait()
    cp_tp.wait()
    cp_in.wait()

    def exp_pass(px, inv):
      @plsc.parallel_loop(0, G_PDF, unroll=4)
      def _(j):
        px[pl.ds(j * L, L)] = jnp.exp(px[pl.ds(j * L, L)]) * inv

    def frame_step(src_a, dst_a, px):
      def tbody(j, accs):
        a0, a1 = accs
        off = j * (2 * L)
        pk0 = pk_v[pl.ds(off, L)]
        pk1 = pk_v[pl.ds(off + L, L)]
        w0, w1 = plsc.unpack(
            tp_v[pl.ds(off, 2 * L)], format=plsc.PackFormat.INTERLEAVED,
            preferred_element_type=jnp.float32)
        s0 = pk0 >> 20
        d0 = (pk0 >> 9) & 2047
        p0 = pk0 & 511
        v0 = (plsc.load_gather(src_a, [s0]) * w0
              * plsc.load_gather(px, [p0]))
        plsc.addupdate_scatter(dst_a, [d0], v0)
        s1 = pk1 >> 20
        d1 = (pk1 >> 9) & 2047
        p1 = pk1 & 511
        v1 = (plsc.load_gather(src_a, [s1]) * w1
              * plsc.load_gather(px, [p1]))
        plsc.addupdate_scatter(dst_a, [d1], v1)
        return (a0 + v0, a1 + v1)

      accs = plsc.parallel_loop(
          0, G_TRANS // 2, unroll=4, carry=(ZERO, ZERO))(tbody)
      c = jnp.sum(accs[0] + accs[1])

      @plsc.parallel_loop(0, G_STATE, unroll=8)
      def _(j):
        src_a[pl.ds(j * L, L)] = ZERO
      return c

    def it(i, carry):
      cvec, inv = carry
      t0 = 2 * i
      pltpu.make_async_copy(x_hbm.at[b, t0], px0, s0).wait()
      exp_pass(px0, inv)
      c0 = frame_step(a_ping, a_pong, px0)
      inv = ONE / lax.broadcast_in_dim(c0, (L,), ())

      @pl.when(i < TFRAMES // 2 - 1)
      def _():
        pltpu.make_async_copy(x_hbm.at[b, t0 + 2], px0, s0).start()

      pltpu.make_async_copy(x_hbm.at[b, t0 + 1], px1, s1).wait()
      exp_pass(px1, inv)
      c1 = frame_step(a_pong, a_ping, px1)
      inv = ONE / lax.broadcast_in_dim(c1, (L,), ())

      @pl.when(i < TFRAMES // 2 - 1)
      def _():
        pltpu.make_async_copy(x_hbm.at[b, t0 + 3], px1, s1).start()

      cvec = jnp.where(lane == (t0 & 15), c0, cvec)
      cvec = jnp.where(lane == ((t0 + 1) & 15), c1, cvec)

      @pl.when((i & 7) == 7)
      def _():
        cs_v[pl.ds((i >> 3) * 16, 16)] = cvec

      return (jnp.where((i & 7) == 7, ONE, cvec), inv)

    cvec, _ = lax.fori_loop(0, TFRAMES // 2, it, (ONE, ONE))
    cs_v[pl.ds((TFRAMES // 16) * 16, L)] = cvec
    pltpu.sync_copy(cs_v, out_hbm.at[b])

  return fwd(x, packed, tp, init)


def _logsum(cs, batch):
  def body(cs_ref, o_ref):
    o_ref[0, 0] = jnp.sum(jnp.log(cs_ref[...])) * (1.0 / batch)

  out = pl.pallas_call(
      body,
      out_shape=jax.ShapeDtypeStruct((1, 1), jnp.float32),
      out_specs=pl.BlockSpec(memory_space=pltpu.SMEM),
  )(cs)
  return out[0, 0]


def kernel(x, transitions, transition_probs, initial_probs):
  src = transitions[:, 0]
  dst = transitions[:, 1]
  pdf = transitions[:, 2]
  packed = (src << 20) | (dst << 9) | pdf
  tp_pairs = (transition_probs.astype(jnp.bfloat16)
              .reshape(-1, 2, 16).transpose(0, 2, 1).reshape(-1))
  cs = _sc_forward(x, packed, tp_pairs, initial_probs)
  return _logsum(cs, x.shape[0])
